# baseline (speedup 1.0000x reference)
import jax
import jax.numpy as jnp
from jax import lax
from jax.experimental import pallas as pl
from jax.experimental.pallas import tpu as pltpu
from jax.experimental.pallas import tpu_sc as plsc

_TOPK = 5
_B = 4
_V = 32
_ROW = 128 * 128 * 3
_PD = 7
_NC = 2
_NS = 16
_NEG = -1e30
_BIG = 1 << 30


def _topk_row(w0, w1, iota):
    idxs, vals = [], []
    for _ in range(_TOPK):
        m0 = jnp.max(w0)
        m1 = jnp.max(w1)
        use0 = m0 >= m1
        i0 = jnp.min(jnp.where(w0 == m0, iota, _BIG))
        i1 = jnp.min(jnp.where(w1 == m1, iota, _BIG))
        idxs.append(jnp.where(use0, i0, i1 + 16))
        vals.append(jnp.where(use0, m0, m1))
        w0 = jnp.where((iota == i0) & use0, _NEG, w0)
        w1 = jnp.where((iota == i1) & jnp.logical_not(use0), _NEG, w1)
    return idxs, vals


def _probs_vec(vals, iota):
    vals_v = jnp.full((16,), _NEG, jnp.float32)
    for t in range(_TOPK):
        vals_v = jnp.where(iota == t, vals[t], vals_v)
    e = jnp.exp(vals_v - vals[0])
    e = jnp.where(iota < _TOPK, e, 0.0)
    return e / jnp.sum(e)


def _body(sel_hbm, img_hbm, pose_hbm, out_img, out_pose, out_prob,
          sel_v, img_v, pose_v, pose_o, prob_o):
    wid = lax.axis_index("s") * _NC + lax.axis_index("c")
    iota = lax.iota(jnp.int32, 16)

    @pl.when(wid < _B * _TOPK)
    def _():
        pltpu.sync_copy(sel_hbm, sel_v)
        b = wid // _TOPK
        k = wid % _TOPK
        w0 = sel_v[pl.ds(b * _V, 16)]
        w1 = sel_v[pl.ds(b * _V + 16, 16)]
        idxs, _ = _topk_row(w0, w1, iota)
        idx_own = idxs[0]
        for t in range(1, _TOPK):
            idx_own = jnp.where(k == t, idxs[t], idx_own)
        g = b * _V + idx_own
        pltpu.sync_copy(img_hbm.at[pl.ds(g * _ROW, _ROW)], img_v)
        pltpu.sync_copy(img_v, out_img.at[pl.ds(wid * _ROW, _ROW)])

    @pl.when(wid == _B * _TOPK)
    def _():
        pltpu.sync_copy(sel_hbm, sel_v)
        pltpu.sync_copy(pose_hbm, pose_v.at[pl.ds(0, _B * _V * _PD)])
        for b in range(_B):
            w0 = sel_v[pl.ds(b * _V, 16)]
            w1 = sel_v[pl.ds(b * _V + 16, 16)]
            idxs, vals = _topk_row(w0, w1, iota)
            prob_o[pl.ds(b * _TOPK, 16)] = _probs_vec(vals, iota)
            for t in range(_TOPK):
                g = b * _V + idxs[t]
                row = pose_v[pl.ds(g * _PD, 16)]
                pose_o[pl.ds((b * _TOPK + t) * _PD, 16)] = row
        pltpu.sync_copy(pose_o.at[pl.ds(0, _B * _TOPK * _PD)], out_pose)
        pltpu.sync_copy(prob_o.at[pl.ds(0, _B * _TOPK)], out_prob)


_sc_call = pl.kernel(
    _body,
    out_type=(
        jax.ShapeDtypeStruct((_B * _TOPK * _ROW,), jnp.float32),
        jax.ShapeDtypeStruct((_B * _TOPK * _PD,), jnp.float32),
        jax.ShapeDtypeStruct((_B * _TOPK,), jnp.float32),
    ),
    mesh=plsc.VectorSubcoreMesh(core_axis_name="c", subcore_axis_name="s"),
    scratch_types=[
        pltpu.VMEM((_B * _V,), jnp.float32),
        pltpu.VMEM((_ROW,), jnp.float32),
        pltpu.VMEM((_B * _V * _PD + 16,), jnp.float32),
        pltpu.VMEM((_B * _TOPK * _PD + 16,), jnp.float32),
        pltpu.VMEM((_B * _TOPK + 16,), jnp.float32),
    ],
    compiler_params=pltpu.CompilerParams(needs_layout_passes=False),
)


@jax.jit
def kernel(selection_weights, images, poses):
    sel = selection_weights.reshape(_B * _V)
    img = images.reshape(_B * _V * _ROW)
    pose = poses.reshape(_B * _V * _PD)
    out_img, out_pose, out_prob = _sc_call(sel, img, pose)
    return (
        out_img.reshape(_B, _TOPK, 128, 128, 3),
        out_pose.reshape(_B, _TOPK, _PD),
        out_prob.reshape(_B, _TOPK),
    )
